# baseline (speedup 1.0000x reference)
import dataclasses
import functools

import jax
import jax.numpy as jnp
from jax import lax
from jax.experimental import pallas as pl
from jax.experimental.pallas import tpu as pltpu
from jax.experimental.pallas import tpu_sc as plsc

_NUM_CORES = 2
_NUM_SUBCORES = 16
_NW = _NUM_CORES * _NUM_SUBCORES
_LANES = 16
_ROW = 8
_G = 4


@functools.lru_cache(maxsize=None)
def _build_sc_lerp_gather(n_start: int, nc: int, two: int, t_total: int):
    assert t_total % _NW == 0 and (nc * two) % _ROW == 0
    w = t_total // _NW
    rows_per_s = nc * two // _ROW
    orows_per_s = t_total * two // _ROW
    ng = n_start // _G
    assert ng % 2 == 0 and w % _LANES == 0
    glen = _G * 2 * w
    olen = _G * w * two // _ROW
    mesh = plsc.VectorSubcoreMesh(core_axis_name="c", subcore_axis_name="s")
    cparams = pltpu.CompilerParams()
    for _f, _v in (("needs_layout_passes", False),
                   ("use_tc_tiling_on_sc", False)):
        if _f in pltpu.CompilerParams.__dataclass_fields__:
            cparams = dataclasses.replace(cparams, **{_f: _v})

    @functools.partial(
        pl.kernel,
        out_type=jax.ShapeDtypeStruct((n_start * orows_per_s, _ROW),
                                      jnp.float32),
        mesh=mesh,
        compiler_params=cparams,
        scratch_types=[
            pltpu.VMEM((w,), jnp.float32),
            pltpu.VMEM((w,), jnp.float32),
            pltpu.VMEM((w,), jnp.int32),
            pltpu.VMEM((w,), jnp.int32),
            pltpu.VMEM((w,), jnp.int32),
            pltpu.VMEM((glen,), jnp.int32),
            pltpu.VMEM((glen,), jnp.int32),
            pltpu.VMEM((glen, _ROW), jnp.float32),
            pltpu.VMEM((glen, _ROW), jnp.float32),
            pltpu.VMEM((olen, _ROW), jnp.float32),
            pltpu.VMEM((olen, _ROW), jnp.float32),
            pltpu.VMEM((olen,), jnp.int32),
            pltpu.VMEM((olen,), jnp.int32),
            pltpu.SemaphoreType.DMA,
            pltpu.SemaphoreType.DMA,
            pltpu.SemaphoreType.DMA,
            pltpu.SemaphoreType.DMA,
        ],
    )
    def sc_kernel(cp_hbm, ts_hbm, out_hbm,
                  ts_v, frac_v, off_v, r0_v, r1_v,
                  gl0, gl1, buf0, buf1, ob0, ob1, ol0, ol1,
                  sg0, sg1, ss0, ss1):
        gl, buf, ob, ol = (gl0, gl1), (buf0, buf1), (ob0, ob1), (ol0, ol1)
        sg, ss = (sg0, sg1), (ss0, ss1)
        wid = lax.axis_index("s") * _NUM_CORES + lax.axis_index("c")
        t0 = wid * w
        pltpu.sync_copy(ts_hbm.at[pl.ds(t0, w)], ts_v)

        @pl.loop(0, w, step=_LANES)
        def _(i):
            tv = ts_v[pl.ds(i, _LANES)]
            idx = (tv * float(nc - 2)).astype(jnp.int32)
            frac_v[pl.ds(i, _LANES)] = (
                tv * float(nc - 1) - idx.astype(jnp.float32))
            off_v[pl.ds(i, _LANES)] = (idx & 3) * 2
            r0 = lax.shift_right_logical(idx, 2)
            r0_v[pl.ds(i, _LANES)] = r0
            r1_v[pl.ds(i, _LANES)] = jnp.minimum(r0 + 1, rows_per_s - 1)

        lane = lax.iota(jnp.int32, _LANES)
        rowsel = lax.shift_right_logical(lane, 1)
        colsel = lax.bitwise_and(lane, 1)
        lanehi = lax.shift_right_logical(lane, 3)
        lanelo = lax.bitwise_and(lane, _ROW - 1)

        def build_gather_list(g, slot):
            bases = [(g * _G + j) * rows_per_s for j in range(_G)]

            @pl.loop(0, w, step=_LANES)
            def _(i):
                r0 = r0_v[pl.ds(i, _LANES)]
                r1 = r1_v[pl.ds(i, _LANES)]
                for j in range(_G):
                    gl[slot][pl.ds(j * 2 * w + i, _LANES)] = r0 + bases[j]
                    gl[slot][pl.ds(j * 2 * w + w + i, _LANES)] = r1 + bases[j]

        def launch_gather(slot):
            pltpu.async_copy(cp_hbm.at[gl[slot]], buf[slot], sg[slot])

        def wait_gather(slot):
            pltpu.make_async_copy(cp_hbm.at[gl[slot]], buf[slot],
                                  sg[slot]).wait()

        def launch_scatter(slot):
            pltpu.async_copy(ob[slot], out_hbm.at[ol[slot]], ss[slot])

        def wait_scatter(slot):
            pltpu.make_async_copy(ob[slot], out_hbm.at[ol[slot]],
                                  ss[slot]).wait()

        def extract(slot):
            @pl.loop(0, w // (_LANES // two))
            def _(k8):
                rows_t = rowsel + k8 * (_LANES // two)
                offv = plsc.load_gather(off_v, [rows_t])
                col_l = offv + colsel
                cross = offv == (_ROW - 2)
                col_r = jnp.where(cross, colsel, col_l + 2)
                radd = jnp.where(cross, w, 0)
                fv = plsc.load_gather(frac_v, [rows_t])
                omf = 1.0 - fv
                rows_o0 = 2 * k8 + lanehi
                for j in range(_G):
                    base = j * 2 * w
                    lv = plsc.load_gather(buf[slot], [rows_t + base, col_l])
                    rv = plsc.load_gather(buf[slot],
                                          [rows_t + base + radd, col_r])
                    ov = omf * lv + fv * rv
                    plsc.store_scatter(
                        ob[slot], [rows_o0 + j * (olen // _G), lanelo], ov)

        def build_scatter_list(g, slot):
            tq = t0 * two // _ROW
            entries_per_j = olen // _G
            @pl.loop(0, olen, step=_LANES)
            def _(e0):
                j = e0 // entries_per_j
                r = e0 % entries_per_j
                s = g * _G + j
                ol[slot][pl.ds(e0, _LANES)] = (
                    s * orows_per_s + tq + r + lane)

        def do_group(g, slot):
            nxt = 1 - slot

            @pl.when(g + 1 < ng)
            def _():
                build_gather_list(g + 1, nxt)
                launch_gather(nxt)

            wait_gather(slot)

            @pl.when(g >= 2)
            def _():
                wait_scatter(slot)

            extract(slot)
            build_scatter_list(g, slot)
            launch_scatter(slot)

        build_gather_list(0, 0)
        launch_gather(0)

        @pl.loop(0, ng, step=2)
        def _(g):
            do_group(g, 0)
            do_group(g + 1, 1)

        wait_scatter(0)
        wait_scatter(1)

    return sc_kernel


def kernel(timestamps, control_points):
    n_start, nc, two = control_points.shape
    t_total = timestamps.shape[0]
    rows_view = control_points.reshape((n_start * nc * two) // _ROW, _ROW)
    sc_kernel = _build_sc_lerp_gather(n_start, nc, two, t_total)
    out_rows = sc_kernel(rows_view, timestamps)
    return out_rows.reshape(n_start, t_total, two)


# speedup vs baseline: 1.0011x; 1.0011x over previous
import dataclasses
import functools

import jax
import jax.numpy as jnp
from jax import lax
from jax.experimental import pallas as pl
from jax.experimental.pallas import tpu as pltpu
from jax.experimental.pallas import tpu_sc as plsc

_NUM_CORES = 2
_NUM_SUBCORES = 16
_NW = _NUM_CORES * _NUM_SUBCORES
_LANES = 16
_ROW = 8
_G = 4


@functools.lru_cache(maxsize=None)
def _build_sc_lerp_gather(n_start: int, nc: int, two: int, t_total: int):
    assert t_total % _NW == 0 and (nc * two) % _ROW == 0
    w = t_total // _NW
    rows_per_s = nc * two // _ROW
    orows_per_s = t_total * two // _ROW
    ng = n_start // _G
    assert ng % 2 == 0 and w % _LANES == 0
    glen = _G * 2 * w
    olen = _G * w * two // _ROW
    mesh = plsc.VectorSubcoreMesh(core_axis_name="c", subcore_axis_name="s")
    cparams = pltpu.CompilerParams()
    for _f, _v in (("needs_layout_passes", False),
                   ("use_tc_tiling_on_sc", False)):
        if _f in pltpu.CompilerParams.__dataclass_fields__:
            cparams = dataclasses.replace(cparams, **{_f: _v})

    @functools.partial(
        pl.kernel,
        out_type=jax.ShapeDtypeStruct((n_start * orows_per_s, _ROW),
                                      jnp.float32),
        mesh=mesh,
        compiler_params=cparams,
        scratch_types=[
            pltpu.VMEM((w,), jnp.float32),
            pltpu.VMEM((w,), jnp.float32),
            pltpu.VMEM((w,), jnp.int32),
            pltpu.VMEM((w,), jnp.int32),
            pltpu.VMEM((w,), jnp.int32),
            pltpu.VMEM((glen,), jnp.int32),
            pltpu.VMEM((glen,), jnp.int32),
            pltpu.VMEM((glen, _ROW), jnp.float32),
            pltpu.VMEM((glen, _ROW), jnp.float32),
            pltpu.VMEM((olen, _ROW), jnp.float32),
            pltpu.VMEM((olen, _ROW), jnp.float32),
            pltpu.VMEM((olen,), jnp.int32),
            pltpu.VMEM((olen,), jnp.int32),
            pltpu.SemaphoreType.DMA,
            pltpu.SemaphoreType.DMA,
            pltpu.SemaphoreType.DMA,
            pltpu.SemaphoreType.DMA,
        ],
    )
    def sc_kernel(cp_hbm, ts_hbm, out_hbm,
                  ts_v, frac_v, off_v, r0_v, r1_v,
                  gl0, gl1, buf0, buf1, ob0, ob1, ol0, ol1,
                  sg0, sg1, ss0, ss1):
        gl, buf, ob, ol = (gl0, gl1), (buf0, buf1), (ob0, ob1), (ol0, ol1)
        sg, ss = (sg0, sg1), (ss0, ss1)
        wid = lax.axis_index("s") * _NUM_CORES + lax.axis_index("c")
        t0 = wid * w
        pltpu.sync_copy(ts_hbm.at[pl.ds(t0, w)], ts_v)

        @pl.loop(0, w, step=_LANES)
        def _(i):
            tv = ts_v[pl.ds(i, _LANES)]
            idx = (tv * float(nc - 2)).astype(jnp.int32)
            frac_v[pl.ds(i, _LANES)] = (
                tv * float(nc - 1) - idx.astype(jnp.float32))
            off_v[pl.ds(i, _LANES)] = (idx & 3) * 2
            r0 = lax.shift_right_logical(idx, 2)
            r0_v[pl.ds(i, _LANES)] = r0
            r1_v[pl.ds(i, _LANES)] = jnp.minimum(r0 + 1, rows_per_s - 1)

        lane = lax.iota(jnp.int32, _LANES)
        rowsel = lax.shift_right_logical(lane, 1)
        colsel = lax.bitwise_and(lane, 1)
        lanehi = lax.shift_right_logical(lane, 3)
        lanelo = lax.bitwise_and(lane, _ROW - 1)

        def build_gather_list(g, slot):
            bases = [(g * _G + j) * rows_per_s for j in range(_G)]

            @pl.loop(0, w, step=_LANES)
            def _(i):
                r0 = r0_v[pl.ds(i, _LANES)]
                r1 = r1_v[pl.ds(i, _LANES)]
                for j in range(_G):
                    gl[slot][pl.ds(j * 2 * w + i, _LANES)] = r0 + bases[j]
                    gl[slot][pl.ds(j * 2 * w + w + i, _LANES)] = r1 + bases[j]

        def launch_gather(slot):
            pltpu.async_copy(cp_hbm.at[gl[slot]], buf[slot], sg[slot])

        def wait_gather(slot):
            pltpu.make_async_copy(cp_hbm.at[gl[slot]], buf[slot],
                                  sg[slot]).wait()

        def launch_scatter(slot):
            pltpu.async_copy(ob[slot], out_hbm.at[ol[slot]], ss[slot])

        def wait_scatter(slot):
            pltpu.make_async_copy(ob[slot], out_hbm.at[ol[slot]],
                                  ss[slot]).wait()

        def extract(slot):
            @pl.loop(0, w // (_LANES // two))
            def _(k8):
                rows_t = rowsel + k8 * (_LANES // two)
                offv = plsc.load_gather(off_v, [rows_t])
                col_l = offv + colsel
                cross = offv == (_ROW - 2)
                col_r = jnp.where(cross, colsel, col_l + 2)
                radd = jnp.where(cross, w, 0)
                fv = plsc.load_gather(frac_v, [rows_t])
                omf = 1.0 - fv
                rows_o0 = 2 * k8 + lanehi
                for j in range(_G):
                    base = j * 2 * w
                    lv = plsc.load_gather(buf[slot], [rows_t + base, col_l])
                    rv = plsc.load_gather(buf[slot],
                                          [rows_t + base + radd, col_r])
                    ov = omf * lv + fv * rv
                    plsc.store_scatter(
                        ob[slot], [rows_o0 + j * (olen // _G), lanelo], ov)

        def write_out_linear(g, slot):
            tq = t0 * two // _ROW
            epj = olen // _G
            for j in range(_G):
                s = g * _G + j
                pltpu.sync_copy(
                    ob[slot].at[pl.ds(j * epj, epj)],
                    out_hbm.at[pl.ds(s * orows_per_s + tq, epj)])

        def do_group(g, slot):
            nxt = 1 - slot

            @pl.when(g + 1 < ng)
            def _():
                build_gather_list(g + 1, nxt)
                launch_gather(nxt)

            wait_gather(slot)
            extract(slot)
            write_out_linear(g, slot)

        build_gather_list(0, 0)
        launch_gather(0)

        @pl.loop(0, ng, step=2)
        def _(g):
            do_group(g, 0)
            do_group(g + 1, 1)

        del launch_scatter, wait_scatter, ol

    return sc_kernel


def kernel(timestamps, control_points):
    n_start, nc, two = control_points.shape
    t_total = timestamps.shape[0]
    rows_view = control_points.reshape((n_start * nc * two) // _ROW, _ROW)
    sc_kernel = _build_sc_lerp_gather(n_start, nc, two, t_total)
    out_rows = sc_kernel(rows_view, timestamps)
    return out_rows.reshape(n_start, t_total, two)


# speedup vs baseline: 1.0015x; 1.0004x over previous
import dataclasses
import functools

import jax
import jax.numpy as jnp
from jax import lax
from jax.experimental import pallas as pl
from jax.experimental.pallas import tpu as pltpu
from jax.experimental.pallas import tpu_sc as plsc

_NUM_CORES = 2
_NUM_SUBCORES = 16
_NW = _NUM_CORES * _NUM_SUBCORES
_LANES = 16
_ROW = 16
_ROWSH = 4
_G = 2


@functools.lru_cache(maxsize=None)
def _build_sc_lerp_gather(n_start: int, nc: int, two: int, t_total: int):
    assert t_total % _NW == 0 and (nc * two) % _ROW == 0
    w = t_total // _NW
    rows_per_s = nc * two // _ROW
    orows_per_s = t_total * two // _ROW
    ng = n_start // _G
    assert ng % 2 == 0 and w % _LANES == 0
    glen = _G * 2 * w
    olen = _G * w * two // _ROW
    mesh = plsc.VectorSubcoreMesh(core_axis_name="c", subcore_axis_name="s")
    cparams = pltpu.CompilerParams()
    for _f, _v in (("needs_layout_passes", False),
                   ("use_tc_tiling_on_sc", False)):
        if _f in pltpu.CompilerParams.__dataclass_fields__:
            cparams = dataclasses.replace(cparams, **{_f: _v})

    @functools.partial(
        pl.kernel,
        out_type=jax.ShapeDtypeStruct((n_start * orows_per_s, _ROW),
                                      jnp.float32),
        mesh=mesh,
        compiler_params=cparams,
        scratch_types=[
            pltpu.VMEM((w,), jnp.float32),
            pltpu.VMEM((w,), jnp.float32),
            pltpu.VMEM((w,), jnp.int32),
            pltpu.VMEM((w,), jnp.int32),
            pltpu.VMEM((w,), jnp.int32),
            pltpu.VMEM((glen,), jnp.int32),
            pltpu.VMEM((glen,), jnp.int32),
            pltpu.VMEM((glen, _ROW), jnp.float32),
            pltpu.VMEM((glen, _ROW), jnp.float32),
            pltpu.VMEM((olen, _ROW), jnp.float32),
            pltpu.VMEM((olen, _ROW), jnp.float32),
            pltpu.VMEM((olen,), jnp.int32),
            pltpu.VMEM((olen,), jnp.int32),
            pltpu.SemaphoreType.DMA,
            pltpu.SemaphoreType.DMA,
            pltpu.SemaphoreType.DMA,
            pltpu.SemaphoreType.DMA,
        ],
    )
    def sc_kernel(cp_hbm, ts_hbm, out_hbm,
                  ts_v, frac_v, off_v, r0_v, r1_v,
                  gl0, gl1, buf0, buf1, ob0, ob1, ol0, ol1,
                  sg0, sg1, ss0, ss1):
        gl, buf, ob, ol = (gl0, gl1), (buf0, buf1), (ob0, ob1), (ol0, ol1)
        sg, ss = (sg0, sg1), (ss0, ss1)
        wid = lax.axis_index("s") * _NUM_CORES + lax.axis_index("c")
        t0 = wid * w
        pltpu.sync_copy(ts_hbm.at[pl.ds(t0, w)], ts_v)

        @pl.loop(0, w, step=_LANES)
        def _(i):
            tv = ts_v[pl.ds(i, _LANES)]
            idx = (tv * float(nc - 2)).astype(jnp.int32)
            frac_v[pl.ds(i, _LANES)] = (
                tv * float(nc - 1) - idx.astype(jnp.float32))
            off_v[pl.ds(i, _LANES)] = (idx & (_ROW // 2 - 1)) * 2
            r0 = lax.shift_right_logical(idx, _ROWSH - 1)
            r0_v[pl.ds(i, _LANES)] = r0
            r1_v[pl.ds(i, _LANES)] = jnp.minimum(r0 + 1, rows_per_s - 1)

        lane = lax.iota(jnp.int32, _LANES)
        rowsel = lax.shift_right_logical(lane, 1)
        colsel = lax.bitwise_and(lane, 1)
        lanehi = lax.shift_right_logical(lane, _ROWSH)
        lanelo = lax.bitwise_and(lane, _ROW - 1)

        def build_gather_list(g, slot):
            bases = [(g * _G + j) * rows_per_s for j in range(_G)]

            @pl.loop(0, w, step=_LANES)
            def _(i):
                r0 = r0_v[pl.ds(i, _LANES)]
                r1 = r1_v[pl.ds(i, _LANES)]
                for j in range(_G):
                    gl[slot][pl.ds(j * 2 * w + i, _LANES)] = r0 + bases[j]
                    gl[slot][pl.ds(j * 2 * w + w + i, _LANES)] = r1 + bases[j]

        def launch_gather(slot):
            pltpu.async_copy(cp_hbm.at[gl[slot]], buf[slot], sg[slot])

        def wait_gather(slot):
            pltpu.make_async_copy(cp_hbm.at[gl[slot]], buf[slot],
                                  sg[slot]).wait()

        def launch_scatter(slot):
            pltpu.async_copy(ob[slot], out_hbm.at[ol[slot]], ss[slot])

        def wait_scatter(slot):
            pltpu.make_async_copy(ob[slot], out_hbm.at[ol[slot]],
                                  ss[slot]).wait()

        def extract(slot):
            @pl.loop(0, w // (_LANES // two))
            def _(k8):
                rows_t = rowsel + k8 * (_LANES // two)
                offv = plsc.load_gather(off_v, [rows_t])
                col_l = offv + colsel
                cross = offv == (_ROW - 2)
                col_r = jnp.where(cross, colsel, col_l + 2)
                radd = jnp.where(cross, w, 0)
                fv = plsc.load_gather(frac_v, [rows_t])
                omf = 1.0 - fv
                rows_o0 = (k8 * _LANES) // _ROW + lanehi
                for j in range(_G):
                    base = j * 2 * w
                    lv = plsc.load_gather(buf[slot], [rows_t + base, col_l])
                    rv = plsc.load_gather(buf[slot],
                                          [rows_t + base + radd, col_r])
                    ov = omf * lv + fv * rv
                    plsc.store_scatter(
                        ob[slot], [rows_o0 + j * (olen // _G), lanelo], ov)

        def write_out_linear(g, slot):
            tq = t0 * two // _ROW
            epj = olen // _G
            for j in range(_G):
                s = g * _G + j
                pltpu.sync_copy(
                    ob[slot].at[pl.ds(j * epj, epj)],
                    out_hbm.at[pl.ds(s * orows_per_s + tq, epj)])

        def do_group(g, slot):
            nxt = 1 - slot

            @pl.when(g + 1 < ng)
            def _():
                build_gather_list(g + 1, nxt)
                launch_gather(nxt)

            wait_gather(slot)
            extract(slot)
            write_out_linear(g, slot)

        build_gather_list(0, 0)
        launch_gather(0)

        @pl.loop(0, ng, step=2)
        def _(g):
            do_group(g, 0)
            do_group(g + 1, 1)

        del launch_scatter, wait_scatter, ol

    return sc_kernel


def kernel(timestamps, control_points):
    n_start, nc, two = control_points.shape
    t_total = timestamps.shape[0]
    rows_view = control_points.reshape((n_start * nc * two) // _ROW, _ROW)
    sc_kernel = _build_sc_lerp_gather(n_start, nc, two, t_total)
    out_rows = sc_kernel(rows_view, timestamps)
    return out_rows.reshape(n_start, t_total, two)


# speedup vs baseline: 48.6710x; 48.5972x over previous
import dataclasses
import functools

import jax
import jax.numpy as jnp
from jax import lax
from jax.experimental import pallas as pl
from jax.experimental.pallas import tpu as pltpu
from jax.experimental.pallas import tpu_sc as plsc

_NUM_CORES = 2
_NUM_SUBCORES = 16
_NW = _NUM_CORES * _NUM_SUBCORES
_LANES = 16
_BLOCK = 128


@functools.lru_cache(maxsize=None)
def _build_sc_lerp_gather(t_total: int, nc: int, d: int):
    assert t_total % (_NW * _BLOCK) == 0
    rows_per_w = t_total // _NW
    nblk = rows_per_w // _BLOCK
    assert nblk % 2 == 0
    mesh = plsc.VectorSubcoreMesh(core_axis_name="c", subcore_axis_name="s")
    cparams = pltpu.CompilerParams()
    if "needs_layout_passes" in pltpu.CompilerParams.__dataclass_fields__:
        cparams = dataclasses.replace(cparams, needs_layout_passes=False)

    @functools.partial(
        pl.kernel,
        out_type=jax.ShapeDtypeStruct((t_total, d), jnp.float32),
        mesh=mesh,
        compiler_params=cparams,
        scratch_types=[
            pltpu.VMEM((rows_per_w,), jnp.float32),
            pltpu.VMEM((rows_per_w,), jnp.float32),
            pltpu.VMEM((_BLOCK,), jnp.int32),
            pltpu.VMEM((_BLOCK,), jnp.int32),
            pltpu.VMEM((_BLOCK,), jnp.int32),
            pltpu.VMEM((_BLOCK,), jnp.int32),
            pltpu.VMEM((_BLOCK, d), jnp.float32),
            pltpu.VMEM((_BLOCK, d), jnp.float32),
            pltpu.VMEM((_BLOCK, d), jnp.float32),
            pltpu.VMEM((_BLOCK, d), jnp.float32),
            pltpu.VMEM((_BLOCK, d), jnp.float32),
            pltpu.SemaphoreType.DMA,
            pltpu.SemaphoreType.DMA,
        ],
    )
    def sc_kernel(table_hbm, ts_hbm, out_hbm,
                  ts_v, frac_v, il0, ir0, il1, ir1,
                  l0, r0, l1, r1, out_v, sg0, sg1):
        il, ir, lv_, rv_ = (il0, il1), (ir0, ir1), (l0, l1), (r0, r1)
        sg = (sg0, sg1)
        wid = lax.axis_index("s") * _NUM_CORES + lax.axis_index("c")
        t0 = wid * rows_per_w
        pltpu.sync_copy(ts_hbm.at[pl.ds(t0, rows_per_w)], ts_v)

        def build_lists(b, slot):
            @pl.loop(0, _BLOCK, step=_LANES)
            def _(i):
                tv = ts_v[pl.ds(b * _BLOCK + i, _LANES)]
                idx = (tv * float(nc - 2)).astype(jnp.int32)
                frac_v[pl.ds(b * _BLOCK + i, _LANES)] = (
                    tv * float(nc - 1) - idx.astype(jnp.float32))
                il[slot][pl.ds(i, _LANES)] = idx
                ir[slot][pl.ds(i, _LANES)] = idx + 1

        def launch(slot):
            pltpu.async_copy(table_hbm.at[il[slot]], lv_[slot], sg[slot])
            pltpu.async_copy(table_hbm.at[ir[slot]], rv_[slot], sg[slot])

        def wait(slot):
            pltpu.make_async_copy(table_hbm.at[il[slot]], lv_[slot],
                                  sg[slot]).wait()
            pltpu.make_async_copy(table_hbm.at[ir[slot]], rv_[slot],
                                  sg[slot]).wait()

        def lerp_and_store(b, slot):
            @pl.loop(0, _BLOCK)
            def _(r):
                fv = plsc.load_gather(
                    frac_v, [jnp.full((_LANES,), b * _BLOCK + r, jnp.int32)])
                omf = 1.0 - fv
                for c in range(0, d, _LANES):
                    lo = lv_[slot][r, pl.ds(c, _LANES)]
                    hi = rv_[slot][r, pl.ds(c, _LANES)]
                    out_v[r, pl.ds(c, _LANES)] = omf * lo + fv * hi

            pltpu.sync_copy(
                out_v, out_hbm.at[pl.ds(t0 + b * _BLOCK, _BLOCK)])

        build_lists(0, 0)
        launch(0)

        @pl.loop(0, nblk, step=2)
        def _(b):
            for off, slot in ((0, 0), (1, 1)):
                bb = b + off

                @pl.when(bb + 1 < nblk)
                def _():
                    build_lists(bb + 1, 1 - slot)
                    launch(1 - slot)

                wait(slot)
                lerp_and_store(bb, slot)

    return sc_kernel


def kernel(timestamps, control_points):
    n_start, nc, two = control_points.shape
    t_total = timestamps.shape[0]
    d = n_start * two
    table = control_points.transpose(1, 0, 2).reshape(nc, d)
    sc_kernel = _build_sc_lerp_gather(t_total, nc, d)
    out_rows = sc_kernel(table, timestamps)
    return out_rows.reshape(t_total, n_start, two).transpose(1, 0, 2)
